# baseline (speedup 1.0000x reference)
import functools

import jax
import jax.numpy as jnp
from jax.experimental import pallas as pl
from jax.experimental.pallas import tpu as pltpu


def _chamfer_body(x1_ref, x2t_ref, d1_ref, d2_ref, racc_ref):
    j = pl.program_id(1)
    nj = pl.num_programs(1)

    x1 = x1_ref[0]
    x2t = x2t_ref[0]

    sq1 = jnp.sum(x1 * x1, axis=1, keepdims=True)
    r0 = x2t[0:1, :]
    r1 = x2t[1:2, :]
    r2 = x2t[2:3, :]
    sq2 = r0 * r0 + r1 * r1 + r2 * r2

    inner = x1[:, 0:1] * r0 + x1[:, 1:2] * r1 + x1[:, 2:3] * r2
    pd = (sq1 + sq2) - 2.0 * inner

    tm = pd.shape[1]
    rp = pd[:, 0:128]
    for k in range(1, tm // 128):
        rp = jnp.minimum(rp, pd[:, k * 128:(k + 1) * 128])

    d2_ref[0, 0] = jnp.min(pd, axis=0)

    @pl.when(j == 0)
    def _():
        racc_ref[...] = rp

    @pl.when(j != 0)
    def _():
        racc_ref[...] = jnp.minimum(racc_ref[...], rp)

    @pl.when(j == nj - 1)
    def _():
        d1_ref[0, 0] = jnp.min(racc_ref[...], axis=1)


@functools.partial(jax.jit, static_argnames=("interpret",))
def _chamfer(xyz1, xyz2, interpret=False):
    B, N, _ = xyz1.shape
    M = xyz2.shape[1]
    TM = 1024

    xyz2t = xyz2.transpose(0, 2, 1)

    grid = (B, M // TM)
    d1, d2 = pl.pallas_call(
        _chamfer_body,
        grid=grid,
        in_specs=[
            pl.BlockSpec((1, N, 3), lambda b, j: (b, 0, 0)),
            pl.BlockSpec((1, 3, TM), lambda b, j: (b, 0, j)),
        ],
        out_specs=[
            pl.BlockSpec((1, 1, N), lambda b, j: (b, 0, 0)),
            pl.BlockSpec((1, 1, TM), lambda b, j: (b, 0, j)),
        ],
        out_shape=[
            jax.ShapeDtypeStruct((B, 1, N), jnp.float32),
            jax.ShapeDtypeStruct((B, 1, M), jnp.float32),
        ],
        scratch_shapes=[pltpu.VMEM((N, 128), jnp.float32)],
        interpret=interpret,
    )(xyz1, xyz2t)
    return d1, d2


def kernel(xyz1, xyz2):
    if xyz1.ndim == 2:
        xyz1 = xyz1[None]
    if xyz2.ndim == 2:
        xyz2 = xyz2[None]
    d1, d2 = _chamfer(xyz1, xyz2)
    return (d1[:, 0, :], d2[:, 0, :])


# speedup vs baseline: 1.6806x; 1.6806x over previous
import functools

import jax
import jax.numpy as jnp
from jax.experimental import pallas as pl
from jax.experimental.pallas import tpu as pltpu


def _chamfer_body(x1_ref, x2_ref, d1_ref, d2_ref, racc_ref):
    j = pl.program_id(1)
    nj = pl.num_programs(1)

    x1 = x1_ref[0]
    x2 = x2_ref[0]

    sq1 = jnp.sum(x1 * x1, axis=1)
    sq2 = jnp.sum(x2 * x2, axis=1)

    inner = jax.lax.dot_general(
        x1, x2,
        dimension_numbers=(((1,), (1,)), ((), ())),
        preferred_element_type=jnp.float32,
    )

    pd = (sq1[:, None] + sq2[None, :]) - 2.0 * inner

    tm = pd.shape[1]
    rp = pd[:, 0:128]
    for k in range(1, tm // 128):
        rp = jnp.minimum(rp, pd[:, k * 128:(k + 1) * 128])

    d2_ref[0, 0] = jnp.min(pd, axis=0)

    @pl.when(j == 0)
    def _():
        racc_ref[...] = rp

    @pl.when(j != 0)
    def _():
        racc_ref[...] = jnp.minimum(racc_ref[...], rp)

    @pl.when(j == nj - 1)
    def _():
        d1_ref[0, 0] = jnp.min(racc_ref[...], axis=1)


@functools.partial(jax.jit, static_argnames=("interpret",))
def _chamfer(xyz1, xyz2, interpret=False):
    B, N, _ = xyz1.shape
    M = xyz2.shape[1]
    TM = 1024

    grid = (B, M // TM)
    d1, d2 = pl.pallas_call(
        _chamfer_body,
        grid=grid,
        in_specs=[
            pl.BlockSpec((1, N, 3), lambda b, j: (b, 0, 0)),
            pl.BlockSpec((1, TM, 3), lambda b, j: (b, j, 0)),
        ],
        out_specs=[
            pl.BlockSpec((1, 1, N), lambda b, j: (b, 0, 0)),
            pl.BlockSpec((1, 1, TM), lambda b, j: (b, 0, j)),
        ],
        out_shape=[
            jax.ShapeDtypeStruct((B, 1, N), jnp.float32),
            jax.ShapeDtypeStruct((B, 1, M), jnp.float32),
        ],
        scratch_shapes=[pltpu.VMEM((N, 128), jnp.float32)],
        interpret=interpret,
    )(xyz1, xyz2)
    return d1, d2


def kernel(xyz1, xyz2):
    if xyz1.ndim == 2:
        xyz1 = xyz1[None]
    if xyz2.ndim == 2:
        xyz2 = xyz2[None]
    d1, d2 = _chamfer(xyz1, xyz2)
    return (d1[:, 0, :], d2[:, 0, :])


# speedup vs baseline: 1.7170x; 1.0217x over previous
import functools

import jax
import jax.numpy as jnp
from jax.experimental import pallas as pl
from jax.experimental.pallas import tpu as pltpu


def _chamfer_body(x1_ref, x2_ref, d1_ref, d2_ref, racc_ref):
    j = pl.program_id(1)
    nj = pl.num_programs(1)

    x1 = x1_ref[0]
    x2 = x2_ref[0]

    sq2 = jnp.sum(x2 * x2, axis=1)
    sq1c = jnp.sum(x1 * x1, axis=1, keepdims=True)

    inner = jax.lax.dot_general(
        x1, x2,
        dimension_numbers=(((1,), (1,)), ((), ())),
        preferred_element_type=jnp.float32,
    )

    pd = (sq1c + sq2[None, :]) - 2.0 * inner

    tm = pd.shape[1]
    rp = pd[:, 0:128]
    for k in range(1, tm // 128):
        rp = jnp.minimum(rp, pd[:, k * 128:(k + 1) * 128])

    d2_ref[0, 0] = jnp.min(pd, axis=0)

    @pl.when(j == 0)
    def _():
        racc_ref[...] = rp

    @pl.when(j != 0)
    def _():
        racc_ref[...] = jnp.minimum(racc_ref[...], rp)

    @pl.when(j == nj - 1)
    def _():
        d1_ref[0, 0] = jnp.min(racc_ref[...], axis=1)


@functools.partial(jax.jit, static_argnames=("interpret",))
def _chamfer(xyz1, xyz2, interpret=False):
    B, N, _ = xyz1.shape
    M = xyz2.shape[1]
    TM = 1024

    grid = (B, M // TM)
    d1, d2 = pl.pallas_call(
        _chamfer_body,
        grid=grid,
        in_specs=[
            pl.BlockSpec((1, N, 3), lambda b, j: (b, 0, 0)),
            pl.BlockSpec((1, TM, 3), lambda b, j: (b, j, 0)),
        ],
        out_specs=[
            pl.BlockSpec((1, 1, N), lambda b, j: (b, 0, 0)),
            pl.BlockSpec((1, 1, TM), lambda b, j: (b, 0, j)),
        ],
        out_shape=[
            jax.ShapeDtypeStruct((B, 1, N), jnp.float32),
            jax.ShapeDtypeStruct((B, 1, M), jnp.float32),
        ],
        scratch_shapes=[pltpu.VMEM((N, 128), jnp.float32)],
        interpret=interpret,
    )(xyz1, xyz2)
    return d1, d2


def kernel(xyz1, xyz2):
    if xyz1.ndim == 2:
        xyz1 = xyz1[None]
    if xyz2.ndim == 2:
        xyz2 = xyz2[None]
    d1, d2 = _chamfer(xyz1, xyz2)
    return (d1[:, 0, :], d2[:, 0, :])
